# baseline (speedup 1.0000x reference)
import functools

import jax
import jax.numpy as jnp
from jax.experimental import pallas as pl
from jax.experimental.pallas import tpu as pltpu

GE = 16
Nt = 4


def _pick_block(n, target):
    if n <= target:
        return n
    for b in range(target, 0, -1):
        if n % b == 0 and (b % 8 == 0 or b == n):
            return b
    return n


def _node_pre_body(x_ref, wa_ref, b1_ref, out_ref):
    out_ref[...] = x_ref[...] @ wa_ref[...] + b1_ref[...]


def _edge_mlp_body(g_ref, ea_ref, wb_ref, w2_ref, b2_ref, out_ref):
    h = jnp.maximum(g_ref[...] + ea_ref[...] @ wb_ref[...], 0.0)
    out_ref[...] = jnp.maximum(h @ w2_ref[...] + b2_ref[...], 0.0)


def _node_post_body(xc_ref, agg_ref, wxc_ref, wagg_ref, b1_ref, w2_ref, b2_ref, out_ref):
    h2 = jnp.maximum(xc_ref[...] @ wxc_ref[...] + agg_ref[...] @ wagg_ref[...] + b1_ref[...], 0.0)
    comb = jnp.maximum(h2 @ w2_ref[...] + b2_ref[...], 0.0)
    out_ref[...] = jnp.concatenate([comb, xc_ref[...][:, GE:]], axis=1)


def _node_post_head_body(xc_ref, agg_ref, wxc_ref, wagg_ref, b1_ref, w2_ref, b2_ref,
                         hw1_ref, hb1_ref, hw2_ref, hb2_ref, out_ref):
    h2 = jnp.maximum(xc_ref[...] @ wxc_ref[...] + agg_ref[...] @ wagg_ref[...] + b1_ref[...], 0.0)
    comb = jnp.maximum(h2 @ w2_ref[...] + b2_ref[...], 0.0)
    h = jnp.maximum(comb @ hw1_ref[...] + hb1_ref[...], 0.0)
    o = jnp.tanh(h @ hw2_ref[...] + hb2_ref[...])
    nor = jnp.sqrt(jnp.sum(o * o, axis=1, keepdims=True))
    out_ref[...] = o / jnp.maximum(1.0, nor)


def _full_spec(shape):
    return pl.BlockSpec(shape, lambda i: (0, 0))


def _rows_spec(b, c):
    return pl.BlockSpec((b, c), lambda i: (i, 0))


def _node_pre(x, wa, b1):
    n = x.shape[0]
    bn = _pick_block(n, 10000)
    return pl.pallas_call(
        _node_pre_body,
        grid=(n // bn,),
        in_specs=[_rows_spec(bn, x.shape[1]), _full_spec(wa.shape), _full_spec(b1.shape)],
        out_specs=_rows_spec(bn, 32),
        out_shape=jax.ShapeDtypeStruct((n, 32), jnp.float32),
    )(x, wa, b1)


def _edge_mlp(g, ea, wb, w2, b2):
    e = g.shape[0]
    be = _pick_block(e, 8000)
    return pl.pallas_call(
        _edge_mlp_body,
        grid=(e // be,),
        in_specs=[_rows_spec(be, 32), _rows_spec(be, ea.shape[1]),
                  _full_spec(wb.shape), _full_spec(w2.shape), _full_spec(b2.shape)],
        out_specs=_rows_spec(be, 32),
        out_shape=jax.ShapeDtypeStruct((e, 32), jnp.float32),
    )(g, ea, wb, w2, b2)


def _node_post(xc, agg, wxc, wagg, b1, w2, b2):
    n = xc.shape[0]
    bn = _pick_block(n, 10000)
    return pl.pallas_call(
        _node_post_body,
        grid=(n // bn,),
        in_specs=[_rows_spec(bn, xc.shape[1]), _rows_spec(bn, 32),
                  _full_spec(wxc.shape), _full_spec(wagg.shape), _full_spec(b1.shape),
                  _full_spec(w2.shape), _full_spec(b2.shape)],
        out_specs=_rows_spec(bn, xc.shape[1]),
        out_shape=jax.ShapeDtypeStruct((n, xc.shape[1]), jnp.float32),
    )(xc, agg, wxc, wagg, b1, w2, b2)


def _node_post_head(xc, agg, wxc, wagg, b1, w2, b2, hw1, hb1, hw2, hb2):
    n = xc.shape[0]
    bn = _pick_block(n, 10000)
    return pl.pallas_call(
        _node_post_head_body,
        grid=(n // bn,),
        in_specs=[_rows_spec(bn, xc.shape[1]), _rows_spec(bn, 32),
                  _full_spec(wxc.shape), _full_spec(wagg.shape), _full_spec(b1.shape),
                  _full_spec(w2.shape), _full_spec(b2.shape),
                  _full_spec(hw1.shape), _full_spec(hb1.shape),
                  _full_spec(hw2.shape), _full_spec(hb2.shape)],
        out_specs=_rows_spec(bn, 2 * Nt),
        out_shape=jax.ShapeDtypeStruct((n, 2 * Nt), jnp.float32),
    )(xc, agg, wxc, wagg, b1, w2, b2, hw1, hb1, hw2, hb2)


def kernel(x, edge_attr, edge_index, m1_W1, m1_b1, m1_W2, m1_b2,
           m2_W1, m2_b1, m2_W2, m2_b2, h_W1, h_b1, h_W2, h_b2):
    n = x.shape[0]
    src = edge_index[0]
    dst = edge_index[1]
    nf = x.shape[1]
    wa = m1_W1[:nf]
    wb = m1_W1[nf:]
    b1r = m1_b1.reshape(1, -1)
    b2r = m1_b2.reshape(1, -1)
    wxc = m2_W1[:nf]
    wagg = m2_W1[nf:]
    m2b1r = m2_b1.reshape(1, -1)
    m2b2r = m2_b2.reshape(1, -1)

    def agg_max(msg):
        a = jax.ops.segment_max(msg, dst, num_segments=n)
        return jnp.where(jnp.isneginf(a), 0.0, a)

    def conv_msgs(xc):
        nx = _node_pre(xc, wa, b1r)
        g = jnp.take(nx, src, axis=0)
        return _edge_mlp(g, edge_attr, wb, m1_W2, b2r)

    agg1 = agg_max(conv_msgs(x))
    x1 = _node_post(x, agg1, wxc, wagg, m2b1r, m2_W2, m2b2r)
    agg2 = agg_max(conv_msgs(x1))
    return _node_post_head(x1, agg2, wxc, wagg, m2b1r, m2_W2, m2b2r,
                           h_W1, h_b1.reshape(1, -1), h_W2, h_b2.reshape(1, -1))


# speedup vs baseline: 1.5907x; 1.5907x over previous
import functools

import jax
import jax.numpy as jnp
from jax import lax
from jax.experimental import pallas as pl
from jax.experimental.pallas import tpu as pltpu
from jax.experimental.pallas import tpu_sc as plsc

GE = 16
Nt = 4

_NC, _NS = 2, 16
_NW = _NC * _NS


def _pick_block(n, target):
    if n <= target:
        return n
    for b in range(target, 0, -1):
        if n % b == 0 and (b % 8 == 0 or b == n):
            return b
    return n


def _node_pre_body(x_ref, wa_ref, b1_ref, out_ref):
    out_ref[...] = x_ref[...] @ wa_ref[...] + b1_ref[...]


def _edge_mlp_body(g_ref, ea_ref, wb_ref, w2_ref, b2_ref, out_ref):
    h = jnp.maximum(g_ref[...] + ea_ref[...] @ wb_ref[...], 0.0)
    out_ref[...] = jnp.maximum(h @ w2_ref[...] + b2_ref[...], 0.0)


def _node_post_body(xc_ref, agg_ref, wxc_ref, wagg_ref, b1_ref, w2_ref, b2_ref, out_ref):
    h2 = jnp.maximum(xc_ref[...] @ wxc_ref[...] + agg_ref[...] @ wagg_ref[...] + b1_ref[...], 0.0)
    comb = jnp.maximum(h2 @ w2_ref[...] + b2_ref[...], 0.0)
    out_ref[...] = jnp.concatenate([comb, xc_ref[...][:, GE:]], axis=1)


def _node_post_head_body(xc_ref, agg_ref, wxc_ref, wagg_ref, b1_ref, w2_ref, b2_ref,
                         hw1_ref, hb1_ref, hw2_ref, hb2_ref, out_ref):
    h2 = jnp.maximum(xc_ref[...] @ wxc_ref[...] + agg_ref[...] @ wagg_ref[...] + b1_ref[...], 0.0)
    comb = jnp.maximum(h2 @ w2_ref[...] + b2_ref[...], 0.0)
    h = jnp.maximum(comb @ hw1_ref[...] + hb1_ref[...], 0.0)
    o = jnp.tanh(h @ hw2_ref[...] + hb2_ref[...])
    nor = jnp.sqrt(jnp.sum(o * o, axis=1, keepdims=True))
    out_ref[...] = o / jnp.maximum(1.0, nor)


def _full_spec(shape):
    return pl.BlockSpec(shape, lambda i: (0, 0))


def _rows_spec(b, c):
    return pl.BlockSpec((b, c), lambda i: (i, 0))


def _node_pre(x, wa, b1):
    n = x.shape[0]
    bn = _pick_block(n, 10000)
    return pl.pallas_call(
        _node_pre_body,
        grid=(n // bn,),
        in_specs=[_rows_spec(bn, x.shape[1]), _full_spec(wa.shape), _full_spec(b1.shape)],
        out_specs=_rows_spec(bn, 32),
        out_shape=jax.ShapeDtypeStruct((n, 32), jnp.float32),
    )(x, wa, b1)


def _edge_mlp(g, ea, wb, w2, b2):
    e = g.shape[0]
    be = _pick_block(e, 8000)
    return pl.pallas_call(
        _edge_mlp_body,
        grid=(e // be,),
        in_specs=[_rows_spec(be, 32), _rows_spec(be, ea.shape[1]),
                  _full_spec(wb.shape), _full_spec(w2.shape), _full_spec(b2.shape)],
        out_specs=_rows_spec(be, 32),
        out_shape=jax.ShapeDtypeStruct((e, 32), jnp.float32),
    )(g, ea, wb, w2, b2)


def _node_post(xc, agg, wxc, wagg, b1, w2, b2):
    n = xc.shape[0]
    bn = _pick_block(n, 10000)
    return pl.pallas_call(
        _node_post_body,
        grid=(n // bn,),
        in_specs=[_rows_spec(bn, xc.shape[1]), _rows_spec(bn, 32),
                  _full_spec(wxc.shape), _full_spec(wagg.shape), _full_spec(b1.shape),
                  _full_spec(w2.shape), _full_spec(b2.shape)],
        out_specs=_rows_spec(bn, xc.shape[1]),
        out_shape=jax.ShapeDtypeStruct((n, xc.shape[1]), jnp.float32),
    )(xc, agg, wxc, wagg, b1, w2, b2)


def _node_post_head(xc, agg, wxc, wagg, b1, w2, b2, hw1, hb1, hw2, hb2):
    n = xc.shape[0]
    bn = _pick_block(n, 10000)
    return pl.pallas_call(
        _node_post_head_body,
        grid=(n // bn,),
        in_specs=[_rows_spec(bn, xc.shape[1]), _rows_spec(bn, 32),
                  _full_spec(wxc.shape), _full_spec(wagg.shape), _full_spec(b1.shape),
                  _full_spec(w2.shape), _full_spec(b2.shape),
                  _full_spec(hw1.shape), _full_spec(hb1.shape),
                  _full_spec(hw2.shape), _full_spec(hb2.shape)],
        out_specs=_rows_spec(bn, 2 * Nt),
        out_shape=jax.ShapeDtypeStruct((n, 2 * Nt), jnp.float32),
    )(xc, agg, wxc, wagg, b1, w2, b2, hw1, hb1, hw2, hb2)


def _sc_gather(table, idx):
    n, d = table.shape
    e = idx.shape[0]
    per = e // _NW
    win = 2000
    chunk = 80
    mesh = plsc.VectorSubcoreMesh(core_axis_name="c", subcore_axis_name="s")

    @functools.partial(
        pl.kernel,
        out_type=jax.ShapeDtypeStruct((e, d), jnp.float32),
        mesh=mesh,
        scratch_types=[
            pltpu.VMEM((win,), jnp.int32),
            pltpu.VMEM((win, d), jnp.float32),
            pltpu.SemaphoreType.DMA,
        ],
        compiler_params=pltpu.CompilerParams(use_tc_tiling_on_sc=False),
    )
    def k(table_hbm, idx_hbm, out_hbm, idx_v, rows_v, sem):
        wid = lax.axis_index("s") * _NC + lax.axis_index("c")
        base = wid * per

        @pl.loop(0, per // win)
        def _win_loop(w):
            b = base + w * win
            pltpu.sync_copy(idx_hbm.at[pl.ds(b, win)], idx_v)
            copies = []
            for j in range(win // chunk):
                copies.append(pltpu.async_copy(
                    table_hbm.at[idx_v.at[pl.ds(j * chunk, chunk)]],
                    rows_v.at[pl.ds(j * chunk, chunk)], sem))
            for c in copies:
                c.wait()
            pltpu.sync_copy(rows_v, out_hbm.at[pl.ds(b, win)])

    return k(table, idx)


def kernel(x, edge_attr, edge_index, m1_W1, m1_b1, m1_W2, m1_b2,
           m2_W1, m2_b1, m2_W2, m2_b2, h_W1, h_b1, h_W2, h_b2):
    n = x.shape[0]
    src = edge_index[0]
    dst = edge_index[1]
    nf = x.shape[1]
    wa = m1_W1[:nf]
    wb = m1_W1[nf:]
    b1r = m1_b1.reshape(1, -1)
    b2r = m1_b2.reshape(1, -1)
    wxc = m2_W1[:nf]
    wagg = m2_W1[nf:]
    m2b1r = m2_b1.reshape(1, -1)
    m2b2r = m2_b2.reshape(1, -1)

    def agg_max(msg):
        a = jax.ops.segment_max(msg, dst, num_segments=n)
        return jnp.where(jnp.isneginf(a), 0.0, a)

    def conv_msgs(xc):
        nx = _node_pre(xc, wa, b1r)
        if n % _NW == 0 and src.shape[0] % (_NW * 2000) == 0:
            g = _sc_gather(nx, src)
        else:
            g = jnp.take(nx, src, axis=0)
        return _edge_mlp(g, edge_attr, wb, m1_W2, b2r)

    agg1 = agg_max(conv_msgs(x))
    x1 = _node_post(x, agg1, wxc, wagg, m2b1r, m2_W2, m2b2r)
    agg2 = agg_max(conv_msgs(x1))
    return _node_post_head(x1, agg2, wxc, wagg, m2b1r, m2_W2, m2b2r,
                           h_W1, h_b1.reshape(1, -1), h_W2, h_b2.reshape(1, -1))


# speedup vs baseline: 1.5917x; 1.0007x over previous
import functools

import jax
import jax.numpy as jnp
from jax import lax
from jax.experimental import pallas as pl
from jax.experimental.pallas import tpu as pltpu
from jax.experimental.pallas import tpu_sc as plsc

GE = 16
Nt = 4

_NC, _NS = 2, 16
_NW = _NC * _NS


def _pick_block(n, target):
    if n <= target:
        return n
    for b in range(target, 0, -1):
        if n % b == 0 and (b % 8 == 0 or b == n):
            return b
    return n


def _node_pre_body(x_ref, wa_ref, b1_ref, out_ref):
    out_ref[...] = x_ref[...] @ wa_ref[...] + b1_ref[...]


def _edge_mlp_body(g_ref, ea_ref, wb_ref, w2_ref, b2_ref, out_ref):
    h = jnp.maximum(g_ref[...] + ea_ref[...] @ wb_ref[...], 0.0)
    out_ref[...] = jnp.maximum(h @ w2_ref[...] + b2_ref[...], 0.0)


def _node_post_body(xc_ref, agg_ref, wxc_ref, wagg_ref, b1_ref, w2_ref, b2_ref, out_ref):
    h2 = jnp.maximum(xc_ref[...] @ wxc_ref[...] + agg_ref[...] @ wagg_ref[...] + b1_ref[...], 0.0)
    comb = jnp.maximum(h2 @ w2_ref[...] + b2_ref[...], 0.0)
    out_ref[...] = jnp.concatenate([comb, xc_ref[...][:, GE:]], axis=1)


def _node_post_head_body(xc_ref, agg_ref, wxc_ref, wagg_ref, b1_ref, w2_ref, b2_ref,
                         hw1_ref, hb1_ref, hw2_ref, hb2_ref, out_ref):
    h2 = jnp.maximum(xc_ref[...] @ wxc_ref[...] + agg_ref[...] @ wagg_ref[...] + b1_ref[...], 0.0)
    comb = jnp.maximum(h2 @ w2_ref[...] + b2_ref[...], 0.0)
    h = jnp.maximum(comb @ hw1_ref[...] + hb1_ref[...], 0.0)
    o = jnp.tanh(h @ hw2_ref[...] + hb2_ref[...])
    nor = jnp.sqrt(jnp.sum(o * o, axis=1, keepdims=True))
    out_ref[...] = o / jnp.maximum(1.0, nor)


def _full_spec(shape):
    return pl.BlockSpec(shape, lambda i: (0, 0))


def _rows_spec(b, c):
    return pl.BlockSpec((b, c), lambda i: (i, 0))


def _node_pre(x, wa, b1):
    n = x.shape[0]
    bn = _pick_block(n, 10000)
    return pl.pallas_call(
        _node_pre_body,
        grid=(n // bn,),
        in_specs=[_rows_spec(bn, x.shape[1]), _full_spec(wa.shape), _full_spec(b1.shape)],
        out_specs=_rows_spec(bn, 32),
        out_shape=jax.ShapeDtypeStruct((n, 32), jnp.float32),
    )(x, wa, b1)


def _edge_mlp(g, ea, wb, w2, b2):
    e = g.shape[0]
    be = _pick_block(e, 8000)
    return pl.pallas_call(
        _edge_mlp_body,
        grid=(e // be,),
        in_specs=[_rows_spec(be, 32), _rows_spec(be, ea.shape[1]),
                  _full_spec(wb.shape), _full_spec(w2.shape), _full_spec(b2.shape)],
        out_specs=_rows_spec(be, 32),
        out_shape=jax.ShapeDtypeStruct((e, 32), jnp.float32),
    )(g, ea, wb, w2, b2)


def _node_post(xc, agg, wxc, wagg, b1, w2, b2):
    n = xc.shape[0]
    bn = _pick_block(n, 10000)
    return pl.pallas_call(
        _node_post_body,
        grid=(n // bn,),
        in_specs=[_rows_spec(bn, xc.shape[1]), _rows_spec(bn, 32),
                  _full_spec(wxc.shape), _full_spec(wagg.shape), _full_spec(b1.shape),
                  _full_spec(w2.shape), _full_spec(b2.shape)],
        out_specs=_rows_spec(bn, xc.shape[1]),
        out_shape=jax.ShapeDtypeStruct((n, xc.shape[1]), jnp.float32),
    )(xc, agg, wxc, wagg, b1, w2, b2)


def _node_post_head(xc, agg, wxc, wagg, b1, w2, b2, hw1, hb1, hw2, hb2):
    n = xc.shape[0]
    bn = _pick_block(n, 10000)
    return pl.pallas_call(
        _node_post_head_body,
        grid=(n // bn,),
        in_specs=[_rows_spec(bn, xc.shape[1]), _rows_spec(bn, 32),
                  _full_spec(wxc.shape), _full_spec(wagg.shape), _full_spec(b1.shape),
                  _full_spec(w2.shape), _full_spec(b2.shape),
                  _full_spec(hw1.shape), _full_spec(hb1.shape),
                  _full_spec(hw2.shape), _full_spec(hb2.shape)],
        out_specs=_rows_spec(bn, 2 * Nt),
        out_shape=jax.ShapeDtypeStruct((n, 2 * Nt), jnp.float32),
    )(xc, agg, wxc, wagg, b1, w2, b2, hw1, hb1, hw2, hb2)


def _sc_gather(table, idx):
    n, d = table.shape
    e = idx.shape[0]
    per = e // _NW
    win = 2000
    chunk = 80
    mesh = plsc.VectorSubcoreMesh(core_axis_name="c", subcore_axis_name="s")

    @functools.partial(
        pl.kernel,
        out_type=jax.ShapeDtypeStruct((e, d), jnp.float32),
        mesh=mesh,
        scratch_types=[
            pltpu.VMEM((win,), jnp.int32),
            pltpu.VMEM((win, d), jnp.float32),
            pltpu.SemaphoreType.DMA,
        ],
        compiler_params=pltpu.CompilerParams(use_tc_tiling_on_sc=False),
    )
    def k(table_hbm, idx_hbm, out_hbm, idx_v, rows_v, sem):
        wid = lax.axis_index("s") * _NC + lax.axis_index("c")
        base = wid * per

        @pl.loop(0, per // win)
        def _win_loop(w):
            b = base + w * win
            pltpu.sync_copy(idx_hbm.at[pl.ds(b, win)], idx_v)
            copies = []
            for j in range(win // chunk):
                copies.append(pltpu.async_copy(
                    table_hbm.at[idx_v.at[pl.ds(j * chunk, chunk)]],
                    rows_v.at[pl.ds(j * chunk, chunk)], sem))
            for c in copies:
                c.wait()
            pltpu.sync_copy(rows_v, out_hbm.at[pl.ds(b, win)])

    return k(table, idx)


def _sc_scatter_max(msg, dst, n):
    e, d = msg.shape
    nb = n // _NW
    dw = 4000
    ch = 64
    nwin = e // dw
    mesh = plsc.VectorSubcoreMesh(core_axis_name="c", subcore_axis_name="s")

    @functools.partial(
        pl.kernel,
        out_type=jax.ShapeDtypeStruct((n * d,), jnp.float32),
        mesh=mesh,
        scratch_types=[
            pltpu.VMEM((2 * dw,), jnp.int32),
            pltpu.VMEM((dw + ch,), jnp.int32),
            pltpu.VMEM((dw + ch,), jnp.int32),
            pltpu.VMEM((nb * 32,), jnp.float32),
            pltpu.VMEM((ch, 32), jnp.float32),
            pltpu.SemaphoreType.DMA,
            pltpu.SemaphoreType.DMA,
        ],
        compiler_params=pltpu.CompilerParams(use_tc_tiling_on_sc=False),
    )
    def k(msg_hbm, dst_hbm, out_hbm, dstw, sel, selo, acc, rows, psem, gsem):
        wid = lax.axis_index("s") * _NC + lax.axis_index("c")
        lo = wid * nb
        iota16 = lax.iota(jnp.int32, 16)

        @pl.loop(0, nb * 32 // 16)
        def _zero_acc(i):
            acc[pl.ds(i * 16, 16)] = jnp.zeros((16,), jnp.float32)

        @pl.loop(0, ch // 16)
        def _zero_sel(i):
            sel[pl.ds(i * 16, 16)] = jnp.zeros((16,), jnp.int32)

        def rmw(base_sel, cnt):
            pltpu.async_copy(msg_hbm.at[sel.at[pl.ds(base_sel, ch)]], rows, gsem).wait()

            @pl.loop(0, cnt)
            def _fold(i):
                o = pl.multiple_of(selo[base_sel + i] * 32, 32)
                isp = jnp.full((16,), i, jnp.int32)
                r0 = plsc.load_gather(rows, [isp, iota16])
                r1 = plsc.load_gather(rows, [isp, iota16 + 16])
                acc[pl.ds(o, 16)] = jnp.maximum(acc[pl.ds(o, 16)], r0)
                acc[pl.ds(o + 16, 16)] = jnp.maximum(acc[pl.ds(o + 16, 16)], r1)

        pltpu.async_copy(dst_hbm.at[pl.ds(0, dw)], dstw.at[pl.ds(0, dw)], psem)

        @pl.loop(0, nwin, init_carry=jnp.int32(0))
        def cursor_fin(w, cursor):
            slot = lax.rem(w, 2)
            soff = pl.multiple_of(slot * dw, dw)
            pltpu.make_async_copy(dst_hbm.at[pl.ds(0, dw)],
                                  dstw.at[pl.ds(0, dw)], psem).wait()

            @pl.when(w + 1 < nwin)
            def _prefetch():
                noff = pl.multiple_of(lax.rem(w + 1, 2) * dw, dw)
                pltpu.async_copy(dst_hbm.at[pl.ds((w + 1) * dw, dw)],
                                 dstw.at[pl.ds(noff, dw)], psem)

            base = w * dw

            @pl.loop(0, dw // 16, init_carry=cursor)
            def cur2(j, cur):
                dv = dstw[pl.ds(soff + j * 16, 16)]
                t = dv - lo
                m = (t >= 0) & (t < nb)
                ids = base + j * 16 + iota16
                inc = plsc.cumsum(m.astype(jnp.int32))
                pos = cur + inc - 1
                plsc.store_scatter(sel, [pos], ids, m)
                plsc.store_scatter(selo, [pos], t, m)
                return cur + inc[15]

            nfull = cur2 // ch

            @pl.loop(0, nfull)
            def _drain(b):
                rmw(pl.multiple_of(b * ch, ch), jnp.int32(ch))

            rem = cur2 - nfull * ch
            src0 = pl.multiple_of(nfull * ch, ch)
            for k2 in range(ch // 16):
                sel[pl.ds(k2 * 16, 16)] = sel[pl.ds(src0 + k2 * 16, 16)]
                selo[pl.ds(k2 * 16, 16)] = selo[pl.ds(src0 + k2 * 16, 16)]
            return rem

        rmw(0, cursor_fin)

        pltpu.sync_copy(acc, out_hbm.at[pl.ds(lo * 32, nb * 32)])

    return k(msg, dst).reshape(n, d)


def kernel(x, edge_attr, edge_index, m1_W1, m1_b1, m1_W2, m1_b2,
           m2_W1, m2_b1, m2_W2, m2_b2, h_W1, h_b1, h_W2, h_b2):
    n = x.shape[0]
    src = edge_index[0]
    dst = edge_index[1]
    nf = x.shape[1]
    wa = m1_W1[:nf]
    wb = m1_W1[nf:]
    b1r = m1_b1.reshape(1, -1)
    b2r = m1_b2.reshape(1, -1)
    wxc = m2_W1[:nf]
    wagg = m2_W1[nf:]
    m2b1r = m2_b1.reshape(1, -1)
    m2b2r = m2_b2.reshape(1, -1)

    def agg_max(msg):
        if n % _NW == 0 and msg.shape[0] % (_NW * 4000) == 0:
            return _sc_scatter_max(msg, dst, n)
        a = jax.ops.segment_max(msg, dst, num_segments=n)
        return jnp.where(jnp.isneginf(a), 0.0, a)

    def conv_msgs(xc):
        nx = _node_pre(xc, wa, b1r)
        if n % _NW == 0 and src.shape[0] % (_NW * 2000) == 0:
            g = _sc_gather(nx, src)
        else:
            g = jnp.take(nx, src, axis=0)
        return _edge_mlp(g, edge_attr, wb, m1_W2, b2r)

    agg1 = agg_max(conv_msgs(x))
    x1 = _node_post(x, agg1, wxc, wagg, m2b1r, m2_W2, m2b2r)
    agg2 = agg_max(conv_msgs(x1))
    return _node_post_head(x1, agg2, wxc, wagg, m2b1r, m2_W2, m2b2r,
                           h_W1, h_b1.reshape(1, -1), h_W2, h_b2.reshape(1, -1))
